# baseline (speedup 1.0000x reference)
import functools

import jax
import jax.numpy as jnp
from jax import lax
from jax.experimental import pallas as pl
from jax.experimental.pallas import tpu as pltpu
from jax.experimental.pallas import tpu_sc as plsc

_VOCAB = 1000000
_EMB = 64
_BATCH = 16384
_C = 3

_NC = 2
_NS = 16
_NW = _NC * _NS
_BPW = _BATCH // _NW
_CB = 64


def _lane_perm(v, idx):
    return lax.gather(
        v, idx.reshape(16, 1),
        lax.GatherDimensionNumbers(
            offset_dims=(), collapsed_slice_dims=(0,), start_index_map=(0,)),
        slice_sizes=(1,),
        mode=lax.GatherScatterMode.PROMISE_IN_BOUNDS)


def _tgather_body(tidx_hbm, ttab_hbm, out_hbm, tidx_v, wrows, sem):
    wid = lax.axis_index("s") * _NC + lax.axis_index("c")
    base = wid * _BPW
    pltpu.sync_copy(tidx_hbm.at[pl.ds(base, _BPW)], tidx_v)

    def trow(g, carry):
        iv = tidx_v[pl.ds(g * 16, 16)]
        for k in range(16):
            pltpu.async_copy(ttab_hbm.at[iv[k]], wrows.at[g * 16 + k], sem)
        return carry

    lax.fori_loop(0, _BPW // 16, trow, 0)
    pltpu.make_async_copy(
        ttab_hbm.at[pl.ds(0, _BPW)], wrows, sem).wait()
    pltpu.sync_copy(wrows, out_hbm.at[pl.ds(base, _BPW)])


def _dots_body(cidx_hbm, ctab_hbm, trows_hbm, out_hbm,
               cidx_v, wrows, crows, out_v, sem):
    wid = lax.axis_index("s") * _NC + lax.axis_index("c")
    base = wid * _BPW
    pltpu.sync_copy(cidx_hbm.at[pl.ds(base * _C, _BPW * _C)], cidx_v)
    pltpu.sync_copy(trows_hbm.at[pl.ds(base, _BPW)], wrows)

    lane = lax.iota(jnp.int32, 16)
    masks = [(lane & sh) != 0 for sh in (1, 2, 4, 8)]
    pidxs = [lane ^ sh for sh in (1, 2, 4, 8)]

    for ck in range(_BPW // _CB):
        cb = ck * _CB
        h = []
        for j in range(_CB * _C // 64):
            h.append(pltpu.async_copy(
                ctab_hbm.at[cidx_v.at[pl.ds(cb * _C + 64 * j, 64)]],
                crows.at[pl.ds(64 * j, 64)], sem))
        for hh in h:
            hh.wait()

        def body(g, carry):
            b0 = g * 16
            for m in range(_C):
                wcache = {}
                prods = []
                for l in range(16):
                    q = m * 16 + l
                    boff, c = q // _C, q % _C
                    if boff not in wcache:
                        wcache[boff] = [
                            wrows[cb + b0 + boff, pl.ds(16 * k, 16)]
                            for k in range(_EMB // 16)]
                    w = wcache[boff]
                    r = (b0 + boff) * _C + c
                    p = w[0] * crows[r, pl.ds(0, 16)]
                    for k in range(1, _EMB // 16):
                        p = p + w[k] * crows[r, pl.ds(16 * k, 16)]
                    prods.append(p)
                vecs = prods
                for step in range(4):
                    msk, pidx = masks[step], pidxs[step]
                    vecs = [jnp.where(msk, vecs[2 * i + 1], vecs[2 * i])
                            + _lane_perm(
                                jnp.where(msk, vecs[2 * i], vecs[2 * i + 1]),
                                pidx)
                            for i in range(len(vecs) // 2)]
                out_v[pl.ds((cb + b0) * _C + m * 16, 16)] = vecs[0]
            return carry

        lax.fori_loop(0, _CB // 16, body, 0)

    pltpu.sync_copy(out_v, out_hbm.at[pl.ds(base * _C, _BPW * _C)])


@jax.jit
def _run(tflat, cflat, ttab, ctab):
    mesh = plsc.VectorSubcoreMesh(core_axis_name="c", subcore_axis_name="s")
    tgather = pl.kernel(
        _tgather_body,
        mesh=mesh,
        out_type=jax.ShapeDtypeStruct((_BATCH, _EMB), jnp.float32),
        scratch_types=[
            pltpu.VMEM((_BPW,), jnp.int32),
            pltpu.VMEM((_BPW, _EMB), jnp.float32),
            pltpu.SemaphoreType.DMA,
        ],
    )
    trows = tgather(tflat, ttab)
    dots = pl.kernel(
        _dots_body,
        mesh=mesh,
        compiler_params=pltpu.CompilerParams(use_tc_tiling_on_sc=False),
        out_type=jax.ShapeDtypeStruct((_BATCH * _C,), jnp.float32),
        scratch_types=[
            pltpu.VMEM((_BPW * _C,), jnp.int32),
            pltpu.VMEM((_BPW, _EMB), jnp.float32),
            pltpu.VMEM((_CB * _C, _EMB), jnp.float32),
            pltpu.VMEM((_BPW * _C,), jnp.float32),
            pltpu.SemaphoreType.DMA,
        ],
    )
    return dots(cflat, ctab, trows).reshape(_BATCH, _C)


def kernel(target, context, target_table, context_table):
    tflat = target.reshape(-1).astype(jnp.int32)
    cflat = context.reshape(-1).astype(jnp.int32)
    return _run(tflat, cflat, target_table, context_table)


# speedup vs baseline: 1.2586x; 1.2586x over previous
import functools

import jax
import jax.numpy as jnp
from jax import lax
from jax.experimental import pallas as pl
from jax.experimental.pallas import tpu as pltpu
from jax.experimental.pallas import tpu_sc as plsc

_VOCAB = 1000000
_EMB = 64
_BATCH = 16384
_C = 3

_NC = 2
_NS = 16
_NW = _NC * _NS
_BPW = _BATCH // _NW
_CB = 64
_NCHUNK = _BPW // _CB


def _lane_perm(v, idx):
    return lax.gather(
        v, idx.reshape(16, 1),
        lax.GatherDimensionNumbers(
            offset_dims=(), collapsed_slice_dims=(0,), start_index_map=(0,)),
        slice_sizes=(1,),
        mode=lax.GatherScatterMode.PROMISE_IN_BOUNDS)


def _asm2vec_body(tidx_hbm, cidx_hbm, ttab_hbm, ctab_hbm, out_hbm,
                  tidx_v, cidx_v, wrows, crows, out_v, *sems):
    wid = lax.axis_index("s") * _NC + lax.axis_index("c")
    base = wid * _BPW

    pltpu.sync_copy(tidx_hbm.at[pl.ds(base, _BPW)], tidx_v)
    pltpu.sync_copy(cidx_hbm.at[pl.ds(base * _C, _BPW * _C)], cidx_v)

    lane = lax.iota(jnp.int32, 16)
    masks = [(lane & sh) != 0 for sh in (1, 2, 4, 8)]
    pidxs = [lane ^ sh for sh in (1, 2, 4, 8)]

    def issue(ck, buf):
        cb = ck * _CB

        def trow(g, carry):
            iv = tidx_v[pl.ds(cb + g * 16, 16)]
            for k in range(16):
                pltpu.async_copy(ttab_hbm.at[iv[k]],
                                 wrows.at[buf, g * 16 + k], sems[buf])
            return carry

        def crow(g, carry):
            iv = cidx_v[pl.ds(cb * _C + g * 16, 16)]
            for k in range(16):
                pltpu.async_copy(ctab_hbm.at[iv[k]],
                                 crows.at[buf, g * 16 + k], sems[buf])
            return carry

        lax.fori_loop(0, _CB // 16, trow, 0)
        lax.fori_loop(0, _CB * _C // 16, crow, 0)

    def drain(buf):
        pltpu.make_async_copy(
            ttab_hbm.at[pl.ds(0, _CB)], wrows.at[buf], sems[buf]).wait()
        pltpu.make_async_copy(
            ctab_hbm.at[pl.ds(0, _CB * _C)], crows.at[buf], sems[buf]).wait()

    issue(0, 0)
    for ck in range(_NCHUNK):
        buf = ck % 2
        if ck + 1 < _NCHUNK:
            issue(ck + 1, (ck + 1) % 2)
        drain(buf)
        cb = ck * _CB

        def body(g, carry):
            b0 = g * 16
            for m in range(_C):
                wcache = {}
                prods = []
                for l in range(16):
                    q = m * 16 + l
                    boff, c = q // _C, q % _C
                    if boff not in wcache:
                        wcache[boff] = [
                            wrows[buf, b0 + boff, pl.ds(16 * k, 16)]
                            for k in range(_EMB // 16)]
                    w = wcache[boff]
                    r = (b0 + boff) * _C + c
                    p = w[0] * crows[buf, r, pl.ds(0, 16)]
                    for k in range(1, _EMB // 16):
                        p = p + w[k] * crows[buf, r, pl.ds(16 * k, 16)]
                    prods.append(p)
                vecs = prods
                for step in range(4):
                    msk, pidx = masks[step], pidxs[step]
                    vecs = [jnp.where(msk, vecs[2 * i + 1], vecs[2 * i])
                            + _lane_perm(
                                jnp.where(msk, vecs[2 * i], vecs[2 * i + 1]),
                                pidx)
                            for i in range(len(vecs) // 2)]
                out_v[pl.ds((cb + b0) * _C + m * 16, 16)] = vecs[0]
            return carry

        lax.fori_loop(0, _CB // 16, body, 0)

    pltpu.sync_copy(out_v, out_hbm.at[pl.ds(base * _C, _BPW * _C)])


@jax.jit
def _run(tflat, cflat, ttab, ctab):
    mesh = plsc.VectorSubcoreMesh(core_axis_name="c", subcore_axis_name="s")
    call = pl.kernel(
        _asm2vec_body,
        mesh=mesh,
        out_type=jax.ShapeDtypeStruct((_BATCH * _C,), jnp.float32),
        scratch_types=[
            pltpu.VMEM((_BPW,), jnp.int32),
            pltpu.VMEM((_BPW * _C,), jnp.int32),
            pltpu.VMEM((2, _CB, _EMB), jnp.float32),
            pltpu.VMEM((2, _CB * _C, _EMB), jnp.float32),
            pltpu.VMEM((_BPW * _C,), jnp.float32),
            pltpu.SemaphoreType.DMA,
            pltpu.SemaphoreType.DMA,
        ],
    )
    return call(tflat, cflat, ttab, ctab).reshape(_BATCH, _C)


def kernel(target, context, target_table, context_table):
    tflat = target.reshape(-1).astype(jnp.int32)
    cflat = context.reshape(-1).astype(jnp.int32)
    return _run(tflat, cflat, target_table, context_table)
